# baseline (speedup 1.0000x reference)
import math

import jax
import jax.numpy as jnp
from jax.experimental import pallas as pl
from jax.experimental.pallas import tpu as pltpu

_D = 128
_HG = 8
_BM = 512
_BQ = 128
_CH = 1024


def _kproj_kernel(x_ref, wk_ref, k_ref, xb_ref, z_ref):
    xv = x_ref[...].astype(jnp.bfloat16)
    xb_ref[...] = xv
    k_ref[...] = jax.lax.dot_general(
        xv, wk_ref[...], (((1,), (1,)), ((), ())),
        preferred_element_type=jnp.float32,
    ).astype(jnp.bfloat16)
    z_ref[...] = jnp.zeros_like(z_ref)


def _scores_kernel(mask_ref, x_ref, xn_ref, wq_ref, k_ref, o_ref, lhs_ref):
    dn = (((1,), (1,)), ((), ()))
    i = pl.program_id(2)

    def build(x_blk):
        qv = jax.lax.dot_general(x_blk, wq_ref[0], dn,
                                 preferred_element_type=jnp.float32
                                 ).astype(jnp.bfloat16)
        rep = jnp.repeat(qv, _HG, axis=0)
        return rep * jnp.tile(mask_ref[...], (_BQ // 2, 1))

    slot = jax.lax.rem(i, 2)

    @pl.when(i == 0)
    def _():
        lhs_ref[0] = build(x_ref[0])

    lhs = lhs_ref[slot]
    kv = k_ref[0]
    S = kv.shape[0]
    for c in range(S // _CH):
        out = jax.lax.dot_general(lhs, kv[c * _CH:(c + 1) * _CH, :], dn,
                                  preferred_element_type=jnp.float32)
        o_ref[0, :, :, c * _CH:(c + 1) * _CH] = out.reshape(_BQ, _HG, _CH)

    lhs_ref[1 - slot] = build(xn_ref[0])


def kernel(x, Wq, Wk, Wv):
    B, S, IN = x.shape
    HID = Wq.shape[0]
    H = HID // _D
    scale = 1.0 / math.sqrt(_D)

    x2 = x.reshape(B * S, IN)
    wqb = (Wq * scale).astype(jnp.bfloat16)
    wkb = Wk.astype(jnp.bfloat16)

    R = B * S
    k2, xb, zeros = pl.pallas_call(
        _kproj_kernel,
        out_shape=(
            jax.ShapeDtypeStruct((R, HID), jnp.bfloat16),
            jax.ShapeDtypeStruct((R, IN), jnp.bfloat16),
            jax.ShapeDtypeStruct((R, HID), jnp.float32),
        ),
        grid=(R // _BM,),
        in_specs=[
            pl.BlockSpec((_BM, IN), lambda i: (i, 0)),
            pl.BlockSpec((HID, IN), lambda i: (0, 0)),
        ],
        out_specs=(
            pl.BlockSpec((_BM, HID), lambda i: (i, 0)),
            pl.BlockSpec((_BM, IN), lambda i: (i, 0)),
            pl.BlockSpec((_BM, HID), lambda i: (i, 0)),
        ),
        compiler_params=pltpu.CompilerParams(
            dimension_semantics=("parallel",),
            vmem_limit_bytes=56 * 1024 * 1024,
        ),
        name="k_proj",
    )(x2, wkb)

    xr = xb.reshape(B, S, IN)
    kr = k2.reshape(B, S, HID)
    wq3 = wqb.reshape(H // _HG, _HG * _D, IN)

    mask16 = (jnp.arange(16, dtype=jnp.int32)[:, None] % _HG
              == jnp.arange(_HG * _D, dtype=jnp.int32)[None, :] // _D
              ).astype(jnp.bfloat16)

    attn_weights = pl.pallas_call(
        _scores_kernel,
        out_shape=jax.ShapeDtypeStruct((B, S, H, S), jnp.float32),
        grid=(B, H // _HG, S // _BQ),
        in_specs=[
            pl.BlockSpec((16, _HG * _D), lambda b, g, i: (0, 0)),
            pl.BlockSpec((1, _BQ, IN), lambda b, g, i: (b, i, 0)),
            pl.BlockSpec((1, _BQ, IN),
                         lambda b, g, i: (b, jnp.minimum(i + 1, S // _BQ - 1), 0)),
            pl.BlockSpec((1, _HG * _D, IN), lambda b, g, i: (g, 0, 0)),
            pl.BlockSpec((1, S, _HG * _D), lambda b, g, i: (b, 0, g)),
        ],
        out_specs=pl.BlockSpec((1, _BQ, _HG, S), lambda b, g, i: (b, i, g, 0)),
        scratch_shapes=[pltpu.VMEM((2, _HG * _BQ, _HG * _D), jnp.bfloat16)],
        compiler_params=pltpu.CompilerParams(
            dimension_semantics=("parallel", "arbitrary", "arbitrary"),
            vmem_limit_bytes=56 * 1024 * 1024,
        ),
        name="qk_scores",
    )(mask16, xr, xr, wq3, kr)

    attn_output = zeros.reshape(B, S, HID)
    return attn_output, attn_weights


# speedup vs baseline: 1.2571x; 1.2571x over previous
import math

import jax
import jax.numpy as jnp
from jax.experimental import pallas as pl
from jax.experimental.pallas import tpu as pltpu

_D = 128
_HG = 8
_BM = 512
_BQ = 256
_CH = 1024


def _proj_kernel(x_ref, wq_ref, wk_ref, q_ref, k_ref, z_ref):
    dn = (((1,), (1,)), ((), ()))
    xv = x_ref[...].astype(jnp.bfloat16)
    q_ref[...] = jax.lax.dot_general(
        xv, wq_ref[...], dn, preferred_element_type=jnp.float32
    ).astype(jnp.bfloat16)
    k_ref[...] = jax.lax.dot_general(
        xv, wk_ref[...], dn, preferred_element_type=jnp.float32
    ).astype(jnp.bfloat16)
    z_ref[...] = jnp.zeros_like(z_ref)


def _scores_kernel(mask_ref, q_ref, k_ref, o_ref):
    qv = q_ref[0]
    kv = k_ref[0]
    rep = jnp.repeat(qv, _HG, axis=0)
    lhs = rep * jnp.tile(mask_ref[...], (_BQ // 2, 1))
    S = kv.shape[0]
    for c in range(S // _CH):
        out = jax.lax.dot_general(lhs, kv[c * _CH:(c + 1) * _CH, :],
                                  (((1,), (1,)), ((), ())),
                                  preferred_element_type=jnp.float32)
        o_ref[0, :, :, c * _CH:(c + 1) * _CH] = out.reshape(_BQ, _HG, _CH)


def kernel(x, Wq, Wk, Wv):
    B, S, IN = x.shape
    HID = Wq.shape[0]
    H = HID // _D
    scale = 1.0 / math.sqrt(_D)

    xb = x.reshape(B * S, IN)
    wqb = (Wq * scale).astype(jnp.bfloat16)
    wkb = Wk.astype(jnp.bfloat16)

    R = B * S
    q2, k2, zeros = pl.pallas_call(
        _proj_kernel,
        out_shape=(
            jax.ShapeDtypeStruct((R, HID), jnp.bfloat16),
            jax.ShapeDtypeStruct((R, HID), jnp.bfloat16),
            jax.ShapeDtypeStruct((R, HID), jnp.float32),
        ),
        grid=(R // _BM,),
        in_specs=[
            pl.BlockSpec((_BM, IN), lambda i: (i, 0)),
            pl.BlockSpec((HID, IN), lambda i: (0, 0)),
            pl.BlockSpec((HID, IN), lambda i: (0, 0)),
        ],
        out_specs=(
            pl.BlockSpec((_BM, HID), lambda i: (i, 0)),
            pl.BlockSpec((_BM, HID), lambda i: (i, 0)),
            pl.BlockSpec((_BM, HID), lambda i: (i, 0)),
        ),
        compiler_params=pltpu.CompilerParams(
            dimension_semantics=("parallel",),
            allow_input_fusion=(False, True, True),
            vmem_limit_bytes=56 * 1024 * 1024,
        ),
        name="qk_proj",
    )(xb, wqb, wkb)

    qr = q2.reshape(B, S, HID)
    kr = k2.reshape(B, S, HID)

    mask16 = (jnp.arange(16, dtype=jnp.int32)[:, None] % _HG
              == jnp.arange(_HG * _D, dtype=jnp.int32)[None, :] // _D
              ).astype(jnp.bfloat16)

    attn_weights = pl.pallas_call(
        _scores_kernel,
        out_shape=jax.ShapeDtypeStruct((B, S, H, S), jnp.float32),
        grid=(B, H // _HG, S // _BQ),
        in_specs=[
            pl.BlockSpec((16, _HG * _D), lambda b, g, i: (0, 0)),
            pl.BlockSpec((1, _BQ, _HG * _D), lambda b, g, i: (b, i, g)),
            pl.BlockSpec((1, S, _HG * _D), lambda b, g, i: (b, 0, g)),
        ],
        out_specs=pl.BlockSpec((1, _BQ, _HG, S), lambda b, g, i: (b, i, g, 0)),
        compiler_params=pltpu.CompilerParams(
            dimension_semantics=("parallel", "arbitrary", "arbitrary"),
            vmem_limit_bytes=56 * 1024 * 1024,
        ),
        name="qk_scores",
    )(mask16, qr, kr)

    attn_output = zeros.reshape(B, S, HID)
    return attn_output, attn_weights
